# baseline (speedup 1.0000x reference)
import functools

import jax
import jax.numpy as jnp
from jax import lax
from jax.experimental import pallas as pl
from jax.experimental.pallas import tpu as pltpu
from jax.experimental.pallas import tpu_sc as plsc

N = 10000
E = 320000
C_IN = 128
CH = 128
H = 64
HH = 32
NC = 2
NS = 16
L = 16
K = 80
NCHUNK = 250
EPT = NCHUNK * K
NB = 5
D = 2
NQ = NCHUNK // NB
RA = 624
REMBASE = NS * RA
REM = N - REMBASE

BN = 2000


def _proj_body(x_ref, w_ref, b2_ref, y_ref, eye_ref):
    z = jnp.dot(x_ref[...], w_ref[...], preferred_element_type=jnp.float32)
    y_ref[0] = z[:, :HH].astype(jnp.bfloat16)
    y_ref[1] = z[:, HH:H].astype(jnp.bfloat16)
    eye_ref[...] = z[:, H:] + b2_ref[...]


def _project(x2d, wcat, b2row):
    return pl.pallas_call(
        _proj_body,
        grid=(N // BN,),
        in_specs=[
            pl.BlockSpec((BN, C_IN), lambda i: (i, 0)),
            pl.BlockSpec((C_IN, C_IN), lambda i: (0, 0)),
            pl.BlockSpec((1, H), lambda i: (0, 0)),
        ],
        out_specs=[
            pl.BlockSpec((2, BN, HH), lambda i: (0, i, 0)),
            pl.BlockSpec((BN, H), lambda i: (i, 0)),
        ],
        out_shape=[
            jax.ShapeDtypeStruct((2, N, HH), jnp.bfloat16),
            jax.ShapeDtypeStruct((N, H), jnp.float32),
        ],
    )(x2d, wcat, b2row)


def _make_scatter(interpret=False):
    mesh = plsc.VectorSubcoreMesh(
        core_axis_name="c", subcore_axis_name="s", num_cores=NC, num_subcores=NS
    )

    @functools.partial(
        pl.kernel,
        out_type=jax.ShapeDtypeStruct((N, H), jnp.float32),
        mesh=mesh,
        interpret=interpret,
        compiler_params=pltpu.CompilerParams(use_tc_tiling_on_sc=False),
        scratch_types=[
            pltpu.VMEM((NCHUNK, K), jnp.int32),
            pltpu.VMEM((NCHUNK, K), jnp.int32),
            pltpu.VMEM((NCHUNK, K), jnp.float32),
            pltpu.VMEM((NB, K, L), jnp.int32),
            pltpu.VMEM((NB, K, HH), jnp.float32),
            pltpu.VMEM((RA, HH), jnp.float32),
            pltpu.VMEM((HH,), jnp.float32),
            pltpu.VMEM_SHARED((N, HH), jnp.float32),
            pltpu.SemaphoreType.DMA,
            pltpu.SemaphoreType.DMA,
        ],
    )
    def scatter(ytab, rows, cols, vals, b1h, out,
                colb, rowb, valb, gring, sring, fbuf, b1v, acc, gsem, ssem):
        c = lax.axis_index("c")
        s = lax.axis_index("s")

        pltpu.sync_copy(cols.at[s], colb)
        pltpu.sync_copy(rows.at[s], rowb)
        pltpu.sync_copy(vals.at[s], valb)
        pltpu.sync_copy(b1h.at[c], b1v)

        bl = b1v[pl.ds(0, L)]
        bh = b1v[pl.ds(L, L)]

        def _init(i, carry):
            fbuf[i, pl.ds(0, L)] = bl
            fbuf[i, pl.ds(L, L)] = bh
            return carry

        lax.fori_loop(0, RA, _init, 0)
        pltpu.sync_copy(fbuf, acc.at[pl.ds(s * RA, RA)])

        @pl.when(s == 0)
        def _():
            pltpu.sync_copy(fbuf.at[pl.ds(0, REM)], acc.at[pl.ds(REMBASE, REM)])

        cN = c * N

        def _off(i, carry):
            for grp in range(K // L):
                sl = (i, pl.ds(grp * L, L))
                colb[sl] = colb[sl] + cN
            return carry

        lax.fori_loop(0, NCHUNK, _off, 0)
        plsc.subcore_barrier()

        def _gissue(g, b):
            pltpu.async_copy(ytab.at[colb.at[g]], gring.at[b], gsem)

        def _gwait(g, b):
            pltpu.make_async_copy(ytab.at[colb.at[g]], gring.at[b], gsem).wait()

        def _sissue(g, b):
            pltpu.async_copy(sring.at[b], acc.at[rowb.at[g]], ssem, add=True)

        def _swait(g, b):
            pltpu.make_async_copy(sring.at[b], acc.at[rowb.at[g]], ssem).wait()

        splats = [jnp.full((L, 1), j, jnp.int32) for j in range(L)]
        dnums = lax.GatherDimensionNumbers(
            offset_dims=(), collapsed_slice_dims=(0,), start_index_map=(0,)
        )

        def _lane_splat(vv, j):
            return lax.gather(vv, splats[j], dnums, (1,),
                              mode=lax.GatherScatterMode.PROMISE_IN_BOUNDS)

        mhi = jnp.full((L,), -65536, jnp.int32)

        def _scale(g, b):
            for grp in range(K // L):
                vv = valb[g, pl.ds(grp * L, L)]
                for j in range(L):
                    k = grp * L + j
                    vk = _lane_splat(vv, j)
                    pw = gring[b, k, pl.ds(0, L)]
                    lo = lax.bitcast_convert_type(pw << 16, jnp.float32)
                    hi = lax.bitcast_convert_type(pw & mhi, jnp.float32)
                    sring[b, k, pl.ds(0, L)] = lo * vk
                    sring[b, k, pl.ds(L, L)] = hi * vk

        for b in range(NB):
            _gissue(b, b)

        def _ring(qq, carry):
            for b in range(NB):
                g = NB * qq + b
                _gwait(g, b)

                @pl.when(g >= NB)
                def _():
                    _swait(g - NB, b)

                _scale(g, b)
                _sissue(g, b)

                @pl.when(g + NB < NCHUNK)
                def _():
                    _gissue(g + NB, b)

            return carry

        lax.fori_loop(0, NQ, _ring, 0)
        for b in range(NB):
            _swait(NCHUNK - NB + b, b)

        plsc.subcore_barrier()
        pltpu.sync_copy(acc.at[pl.ds(s * RA, RA)], fbuf)
        pltpu.sync_copy(fbuf, out.at[pl.ds(s * RA, RA), pl.ds(c * HH, HH)])

        @pl.when(s == 0)
        def _():
            pltpu.sync_copy(acc.at[pl.ds(REMBASE, REM)], fbuf.at[pl.ds(0, REM)])
            pltpu.sync_copy(fbuf.at[pl.ds(0, REM)],
                            out.at[pl.ds(REMBASE, REM), pl.ds(c * HH, HH)])

    return scatter


_scatter = _make_scatter()


_W1_PERM = jnp.array(
    [h * HH + (p // 2) + (p % 2) * L for h in range(2) for p in range(HH)],
    dtype=jnp.int32,
)


def kernel(x, edge_index, edge_vals, W1, b1, W2, b2):
    x2d = x[0]
    wcat = jnp.concatenate([W1[_W1_PERM], W2], axis=0).T
    b2row = b2[None, :]
    ytab, eye = _project(x2d, wcat, b2row)
    ytab2 = lax.bitcast_convert_type(
        ytab.reshape(NC * N, L, 2), jnp.int32)
    rows3d = edge_index[0].astype(jnp.int32).reshape(NS, NCHUNK, K)
    cols3d = edge_index[1].astype(jnp.int32).reshape(NS, NCHUNK, K)
    vals3d = edge_vals.reshape(NS, NCHUNK, K)
    b1h = b1.reshape(NC, HH)
    lin = _scatter(ytab2, rows3d, cols3d, vals3d, b1h)
    out = jnp.concatenate([lin, eye], axis=1)
    return out[None]


# speedup vs baseline: 1.3125x; 1.3125x over previous
import functools

import jax
import jax.numpy as jnp
from jax import lax
from jax.experimental import pallas as pl
from jax.experimental.pallas import tpu as pltpu
from jax.experimental.pallas import tpu_sc as plsc

N = 10000
E = 320000
C_IN = 128
CH = 128
H = 64
HH = 32
NC = 2
NS = 16
L = 16
K = 80
NCHUNK = 250
EPT = NCHUNK * K
NB = 5
D = 2
NQ = NCHUNK // NB
RA = 624
REMBASE = NS * RA
REM = N - REMBASE

BN = 2000


def _proj_body(x_ref, w_ref, b2_ref, y_ref, eye_ref):
    z = jnp.dot(x_ref[...], w_ref[...], preferred_element_type=jnp.float32)
    y_ref[0] = z[:, :HH]
    y_ref[1] = z[:, HH:H]
    eye_ref[...] = z[:, H:] + b2_ref[...]


def _project(x2d, wcat, b2row):
    return pl.pallas_call(
        _proj_body,
        grid=(N // BN,),
        in_specs=[
            pl.BlockSpec((BN, C_IN), lambda i: (i, 0)),
            pl.BlockSpec((C_IN, C_IN), lambda i: (0, 0)),
            pl.BlockSpec((1, H), lambda i: (0, 0)),
        ],
        out_specs=[
            pl.BlockSpec((2, BN, HH), lambda i: (0, i, 0)),
            pl.BlockSpec((BN, H), lambda i: (i, 0)),
        ],
        out_shape=[
            jax.ShapeDtypeStruct((2, N, HH), jnp.float32),
            jax.ShapeDtypeStruct((N, H), jnp.float32),
        ],
    )(x2d, wcat, b2row)


def _make_scatter(interpret=False):
    mesh = plsc.VectorSubcoreMesh(
        core_axis_name="c", subcore_axis_name="s", num_cores=NC, num_subcores=NS
    )

    @functools.partial(
        pl.kernel,
        out_type=jax.ShapeDtypeStruct((N, H), jnp.float32),
        mesh=mesh,
        interpret=interpret,
        compiler_params=pltpu.CompilerParams(use_tc_tiling_on_sc=False),
        scratch_types=[
            pltpu.VMEM((NCHUNK, K), jnp.int32),
            pltpu.VMEM((NCHUNK, K), jnp.int32),
            pltpu.VMEM((NCHUNK, K), jnp.float32),
            pltpu.VMEM((NB, K, HH), jnp.float32),
            pltpu.VMEM((NB, K, HH), jnp.float32),
            pltpu.VMEM((RA, HH), jnp.float32),
            pltpu.VMEM((HH,), jnp.float32),
            pltpu.VMEM_SHARED((N, HH), jnp.float32),
            pltpu.SemaphoreType.DMA,
            pltpu.SemaphoreType.DMA,
        ],
    )
    def scatter(ytab, rows, cols, vals, b1h, out,
                colb, rowb, valb, gring, sring, fbuf, b1v, acc, gsem, ssem):
        c = lax.axis_index("c")
        s = lax.axis_index("s")

        pltpu.sync_copy(cols.at[s], colb)
        pltpu.sync_copy(rows.at[s], rowb)
        pltpu.sync_copy(vals.at[s], valb)
        pltpu.sync_copy(b1h.at[c], b1v)

        bl = b1v[pl.ds(0, L)]
        bh = b1v[pl.ds(L, L)]

        def _init(i, carry):
            fbuf[i, pl.ds(0, L)] = bl
            fbuf[i, pl.ds(L, L)] = bh
            return carry

        lax.fori_loop(0, RA, _init, 0)
        pltpu.sync_copy(fbuf, acc.at[pl.ds(s * RA, RA)])

        @pl.when(s == 0)
        def _():
            pltpu.sync_copy(fbuf.at[pl.ds(0, REM)], acc.at[pl.ds(REMBASE, REM)])

        cN = c * N

        def _off(i, carry):
            for grp in range(K // L):
                sl = (i, pl.ds(grp * L, L))
                colb[sl] = colb[sl] + cN
            return carry

        lax.fori_loop(0, NCHUNK, _off, 0)
        plsc.subcore_barrier()

        def _gissue(g, b):
            pltpu.async_copy(ytab.at[colb.at[g]], gring.at[b], gsem)

        def _gwait(g, b):
            pltpu.make_async_copy(ytab.at[colb.at[g]], gring.at[b], gsem).wait()

        def _sissue(g, b):
            pltpu.async_copy(sring.at[b], acc.at[rowb.at[g]], ssem, add=True)

        def _swait(g, b):
            pltpu.make_async_copy(sring.at[b], acc.at[rowb.at[g]], ssem).wait()

        splats = [jnp.full((L, 1), j, jnp.int32) for j in range(L)]
        dnums = lax.GatherDimensionNumbers(
            offset_dims=(), collapsed_slice_dims=(0,), start_index_map=(0,)
        )

        def _lane_splat(vv, j):
            return lax.gather(vv, splats[j], dnums, (1,),
                              mode=lax.GatherScatterMode.PROMISE_IN_BOUNDS)

        def _scale(g, b):
            for grp in range(K // L):
                vv = valb[g, pl.ds(grp * L, L)]
                for j in range(L):
                    k = grp * L + j
                    vk = _lane_splat(vv, j)
                    sring[b, k, pl.ds(0, L)] = gring[b, k, pl.ds(0, L)] * vk
                    sring[b, k, pl.ds(L, L)] = gring[b, k, pl.ds(L, L)] * vk

        for b in range(NB):
            _gissue(b, b)

        def _ring(qq, carry):
            for b in range(NB):
                g = NB * qq + b
                _gwait(g, b)

                @pl.when(g >= NB)
                def _():
                    _swait(g - NB, b)

                _scale(g, b)
                _sissue(g, b)

                @pl.when(g + NB < NCHUNK)
                def _():
                    _gissue(g + NB, b)

            return carry

        lax.fori_loop(0, NQ, _ring, 0)
        for b in range(NB):
            _swait(NCHUNK - NB + b, b)

        plsc.subcore_barrier()
        pltpu.sync_copy(acc.at[pl.ds(s * RA, RA)], fbuf)
        pltpu.sync_copy(fbuf, out.at[pl.ds(s * RA, RA), pl.ds(c * HH, HH)])

        @pl.when(s == 0)
        def _():
            pltpu.sync_copy(acc.at[pl.ds(REMBASE, REM)], fbuf.at[pl.ds(0, REM)])
            pltpu.sync_copy(fbuf.at[pl.ds(0, REM)],
                            out.at[pl.ds(REMBASE, REM), pl.ds(c * HH, HH)])

    return scatter


_scatter = _make_scatter()


_W1_PERM = jnp.array(
    [h * HH + (p // 2) + (p % 2) * L for h in range(2) for p in range(HH)],
    dtype=jnp.int32,
)


def kernel(x, edge_index, edge_vals, W1, b1, W2, b2):
    x2d = x[0]
    wcat = jnp.concatenate([W1, W2], axis=0).T
    b2row = b2[None, :]
    ytab, eye = _project(x2d, wcat, b2row)
    ytab2 = ytab.reshape(NC * N, HH)
    rows3d = edge_index[0].astype(jnp.int32).reshape(NS, NCHUNK, K)
    cols3d = edge_index[1].astype(jnp.int32).reshape(NS, NCHUNK, K)
    vals3d = edge_vals.reshape(NS, NCHUNK, K)
    b1h = b1.reshape(NC, HH)
    lin = _scatter(ytab2, rows3d, cols3d, vals3d, b1h)
    out = jnp.concatenate([lin, eye], axis=1)
    return out[None]
